# baseline (speedup 1.0000x reference)
import functools

import jax
import jax.numpy as jnp
from jax import lax
from jax.experimental import pallas as pl
from jax.experimental.pallas import tpu as pltpu
from jax.experimental.pallas import tpu_sc as plsc

B = 4096
L = 200
E = 64
W = E // 2
VOCAB = 100000

NC = 2
NS = 16
NW = NC * NS

ROWS_PER_W = B // NW
CH = 4
N_CHUNKS = ROWS_PER_W // CH
HALF = L // 2
TOK_CH = CH * L

VB = 1024
NVB = (VOCAB + VB - 1) // VB
VPAD = NVB * VB


def _conv_body(tt_ref, o_ref):
    bound = VOCAB - pl.program_id(0) * VB
    col = lax.broadcasted_iota(jnp.int32, (E, VB), 1)
    x = jnp.where(col < bound, tt_ref[...], 0.0)
    t = lax.bitcast_convert_type(x.T, jnp.int32)
    rnd = jnp.int32(0x8000)
    lo = lax.shift_right_logical(t[:, :W] + rnd, 16)
    hi = lax.bitwise_and(t[:, W:] + rnd, jnp.int32(-65536))
    words = lax.bitwise_or(hi, lo)
    for m in range(4):
        o_ref[:, pl.ds(m * W, W)] = words[m * (VB // 4):(m + 1) * (VB // 4), :]


def _convert_table(table_t):
    packed = pl.pallas_call(
        _conv_body,
        grid=(NVB,),
        in_specs=[pl.BlockSpec((E, VB), lambda i: (0, i))],
        out_specs=pl.BlockSpec((VB // 4, 2 * E), lambda i: (i, 0)),
        out_shape=jax.ShapeDtypeStruct((VPAD // 4, 2 * E), jnp.int32),
    )(table_t)
    return packed.reshape(VPAD, W)


_PERM = (list(range(0, 16)) + list(range(32, 48))
         + list(range(16, 32)) + list(range(48, 64)))


GGRP = 80
NGRP = TOK_CH // GGRP
TOK_W = ROWS_PER_W * L


def _pool_body(texts_hbm, table_hbm, out_hbm, idx_v, rows_v, acc_v,
               sem0, sem1):
    wid = lax.axis_index("s") * NC + lax.axis_index("c")
    sems = (sem0, sem1)

    pltpu.sync_copy(texts_hbm.at[pl.ds(wid * TOK_W, TOK_W)], idx_v)

    def remap_body(t, carry):
        v = idx_v[pl.ds(16 * t, 16)]
        r = (lax.bitwise_and(v, jnp.int32(-1024))
             | lax.shift_left(lax.bitwise_and(v, jnp.int32(255)), 2)
             | lax.bitwise_and(lax.shift_right_logical(v, 8), jnp.int32(3)))
        idx_v[pl.ds(16 * t, 16)] = r
        return carry

    lax.fori_loop(0, TOK_W // 16, remap_body, 0)

    def stage(s, g):
        for j in range(NGRP):
            pltpu.async_copy(
                table_hbm.at[idx_v.at[pl.ds(g * TOK_CH + j * GGRP, GGRP)]],
                rows_v.at[s, pl.ds(j * GGRP, GGRP)],
                sems[s])

    def drain(s):
        pltpu.make_async_copy(
            table_hbm.at[pl.ds(0, TOK_CH)], rows_v.at[s], sems[s]).wait()

    def reduce_store(s, g):
        row_base = wid * ROWS_PER_W + g * CH
        for r in range(CH):
            def tok_body(t, acc, r=r):
                new = list(acc)
                for u in range(2):
                    base = r * L + 2 * t + u
                    for h in range(2):
                        w = rows_v[s, base, pl.ds(16 * h, 16)]
                        ev = plsc.bitcast(lax.shift_left(w, 16), jnp.float32)
                        od = plsc.bitcast(
                            lax.bitwise_and(w, jnp.int32(-65536)), jnp.float32)
                        new[2 * h] = new[2 * h] + ev
                        new[2 * h + 1] = new[2 * h + 1] + od
                return tuple(new)
            acc = lax.fori_loop(
                0, L // 2, tok_body,
                tuple(jnp.zeros((16,), jnp.float32) for _ in range(4)))
            for c in range(4):
                acc_v[r, pl.ds(c * 16, 16)] = acc[c] * (1.0 / L)
        pltpu.sync_copy(acc_v, out_hbm.at[pl.ds(row_base, CH)])

    stage(0, 0)

    def pair_body(i, carry):
        g0 = 2 * i
        stage(1, g0 + 1)
        drain(0)
        reduce_store(0, g0)

        @pl.when(g0 + 2 < N_CHUNKS)
        def _():
            stage(0, g0 + 2)

        drain(1)
        reduce_store(1, g0 + 1)
        return carry

    lax.fori_loop(0, N_CHUNKS // 2, pair_body, 0)


_pool = functools.partial(
    pl.kernel,
    out_type=jax.ShapeDtypeStruct((B, E), jnp.float32),
    mesh=plsc.VectorSubcoreMesh(core_axis_name="c", subcore_axis_name="s"),
    compiler_params=pltpu.CompilerParams(use_tc_tiling_on_sc=False,
                                         needs_layout_passes=False),
    scratch_types=[
        pltpu.VMEM((TOK_W,), jnp.int32),
        pltpu.VMEM((2, TOK_CH, W), jnp.int32),
        pltpu.VMEM((CH, E), jnp.float32),
        pltpu.SemaphoreType.DMA,
        pltpu.SemaphoreType.DMA,
    ],
)(_pool_body)


def _mlp_body(pt_ref, w1t_ref, b1_ref, w2t_ref, b2_ref, ot_ref):
    ht = jnp.dot(w1t_ref[...], pt_ref[...], preferred_element_type=jnp.float32)
    ht = jnp.maximum(ht + b1_ref[...], 0.0)
    ot_ref[...] = (jnp.dot(w2t_ref[...], ht, preferred_element_type=jnp.float32)
                   + b2_ref[...])


def _mlp_t(pooled_t, W1t, b1c, W2t, b2c):
    BM = 512
    H = W1t.shape[0]
    C = W2t.shape[0]
    return pl.pallas_call(
        _mlp_body,
        grid=(B // BM,),
        in_specs=[
            pl.BlockSpec((E, BM), lambda i: (0, i)),
            pl.BlockSpec((H, E), lambda i: (0, 0)),
            pl.BlockSpec((H, 1), lambda i: (0, 0)),
            pl.BlockSpec((C, H), lambda i: (0, 0)),
            pl.BlockSpec((C, 1), lambda i: (0, 0)),
        ],
        out_specs=pl.BlockSpec((C, BM), lambda i: (0, i)),
        out_shape=jax.ShapeDtypeStruct((C, B), jnp.float32),
    )(pooled_t, W1t, b1c, W2t, b2c)


def kernel(texts, table, W1, b1, W2, b2):
    texts2 = texts.reshape(-1).astype(jnp.int32)
    pooled_p = _pool(texts2, _convert_table(table.T))
    W1tp = W1.T[:, jnp.array(_PERM)]
    out_t = _mlp_t(pooled_p.T, W1tp, b1.reshape(-1, 1), W2.T, b2.reshape(-1, 1))
    return out_t.T


# speedup vs baseline: 1.0019x; 1.0019x over previous
import functools

import jax
import jax.numpy as jnp
from jax import lax
from jax.experimental import pallas as pl
from jax.experimental.pallas import tpu as pltpu
from jax.experimental.pallas import tpu_sc as plsc

B = 4096
L = 200
E = 64
W = E // 2
VOCAB = 100000

NC = 2
NS = 16
NW = NC * NS

ROWS_PER_W = B // NW
CH = 4
N_CHUNKS = ROWS_PER_W // CH
HALF = L // 2
TOK_CH = CH * L

VB = 1024
NVB = (VOCAB + VB - 1) // VB
VPAD = NVB * VB


def _conv_body(tt_ref, o_ref):
    bound = VOCAB - pl.program_id(0) * VB
    col = lax.broadcasted_iota(jnp.int32, (E, VB), 1)
    x = jnp.where(col < bound, tt_ref[...], 0.0)
    t = lax.bitcast_convert_type(x.T, jnp.int32)
    rnd = jnp.int32(0x8000)
    lo = lax.shift_right_logical(t[:, :W] + rnd, 16)
    hi = lax.bitwise_and(t[:, W:] + rnd, jnp.int32(-65536))
    words = lax.bitwise_or(hi, lo)
    for m in range(4):
        o_ref[:, pl.ds(m * W, W)] = words[m * (VB // 4):(m + 1) * (VB // 4), :]


def _convert_table(table_t):
    packed = pl.pallas_call(
        _conv_body,
        grid=(NVB,),
        in_specs=[pl.BlockSpec((E, VB), lambda i: (0, i))],
        out_specs=pl.BlockSpec((VB // 4, 2 * E), lambda i: (i, 0)),
        out_shape=jax.ShapeDtypeStruct((VPAD // 4, 2 * E), jnp.int32),
    )(table_t)
    return packed.reshape(VPAD, W)


_PERM = (list(range(0, 16)) + list(range(32, 48))
         + list(range(16, 32)) + list(range(48, 64)))


GGRP = 40
NGRP = TOK_CH // GGRP
TOK_W = ROWS_PER_W * L


def _pool_body(texts_hbm, table_hbm, out_hbm, idx_v, rows_v, acc_v,
               sem0, sem1):
    wid = lax.axis_index("s") * NC + lax.axis_index("c")
    sems = (sem0, sem1)

    pltpu.sync_copy(texts_hbm.at[pl.ds(wid * TOK_W, TOK_W)], idx_v)

    def remap_body(t, carry):
        v = idx_v[pl.ds(16 * t, 16)]
        r = (lax.bitwise_and(v, jnp.int32(-1024))
             | lax.shift_left(lax.bitwise_and(v, jnp.int32(255)), 2)
             | lax.bitwise_and(lax.shift_right_logical(v, 8), jnp.int32(3)))
        idx_v[pl.ds(16 * t, 16)] = r
        return carry

    lax.fori_loop(0, TOK_W // 16, remap_body, 0)

    def stage(s, g):
        for j in range(NGRP):
            pltpu.async_copy(
                table_hbm.at[idx_v.at[pl.ds(g * TOK_CH + j * GGRP, GGRP)]],
                rows_v.at[s, pl.ds(j * GGRP, GGRP)],
                sems[s])

    def drain(s):
        pltpu.make_async_copy(
            table_hbm.at[pl.ds(0, TOK_CH)], rows_v.at[s], sems[s]).wait()

    def reduce_store(s, g):
        row_base = wid * ROWS_PER_W + g * CH
        for r in range(CH):
            def tok_body(t, acc, r=r):
                new = list(acc)
                for u in range(2):
                    base = r * L + 2 * t + u
                    for h in range(2):
                        w = rows_v[s, base, pl.ds(16 * h, 16)]
                        ev = plsc.bitcast(lax.shift_left(w, 16), jnp.float32)
                        od = plsc.bitcast(
                            lax.bitwise_and(w, jnp.int32(-65536)), jnp.float32)
                        new[2 * h] = new[2 * h] + ev
                        new[2 * h + 1] = new[2 * h + 1] + od
                return tuple(new)
            acc = lax.fori_loop(
                0, L // 2, tok_body,
                tuple(jnp.zeros((16,), jnp.float32) for _ in range(4)))
            for c in range(4):
                acc_v[r, pl.ds(c * 16, 16)] = acc[c] * (1.0 / L)
        pltpu.sync_copy(acc_v, out_hbm.at[pl.ds(row_base, CH)])

    stage(0, 0)

    def pair_body(i, carry):
        g0 = 2 * i
        stage(1, g0 + 1)
        drain(0)
        reduce_store(0, g0)

        @pl.when(g0 + 2 < N_CHUNKS)
        def _():
            stage(0, g0 + 2)

        drain(1)
        reduce_store(1, g0 + 1)
        return carry

    lax.fori_loop(0, N_CHUNKS // 2, pair_body, 0)


_pool = functools.partial(
    pl.kernel,
    out_type=jax.ShapeDtypeStruct((B, E), jnp.float32),
    mesh=plsc.VectorSubcoreMesh(core_axis_name="c", subcore_axis_name="s"),
    compiler_params=pltpu.CompilerParams(use_tc_tiling_on_sc=False,
                                         needs_layout_passes=False),
    scratch_types=[
        pltpu.VMEM((TOK_W,), jnp.int32),
        pltpu.VMEM((2, TOK_CH, W), jnp.int32),
        pltpu.VMEM((CH, E), jnp.float32),
        pltpu.SemaphoreType.DMA,
        pltpu.SemaphoreType.DMA,
    ],
)(_pool_body)


def _mlp_body(pt_ref, w1t_ref, b1_ref, w2t_ref, b2_ref, ot_ref):
    ht = jnp.dot(w1t_ref[...], pt_ref[...], preferred_element_type=jnp.float32)
    ht = jnp.maximum(ht + b1_ref[...], 0.0)
    ot_ref[...] = (jnp.dot(w2t_ref[...], ht, preferred_element_type=jnp.float32)
                   + b2_ref[...])


def _mlp_t(pooled_t, W1t, b1c, W2t, b2c):
    BM = 512
    H = W1t.shape[0]
    C = W2t.shape[0]
    return pl.pallas_call(
        _mlp_body,
        grid=(B // BM,),
        in_specs=[
            pl.BlockSpec((E, BM), lambda i: (0, i)),
            pl.BlockSpec((H, E), lambda i: (0, 0)),
            pl.BlockSpec((H, 1), lambda i: (0, 0)),
            pl.BlockSpec((C, H), lambda i: (0, 0)),
            pl.BlockSpec((C, 1), lambda i: (0, 0)),
        ],
        out_specs=pl.BlockSpec((C, BM), lambda i: (0, i)),
        out_shape=jax.ShapeDtypeStruct((C, B), jnp.float32),
    )(pooled_t, W1t, b1c, W2t, b2c)


def kernel(texts, table, W1, b1, W2, b2):
    texts2 = texts.reshape(-1).astype(jnp.int32)
    pooled_p = _pool(texts2, _convert_table(table.T))
    W1tp = W1.T[:, jnp.array(_PERM)]
    out_t = _mlp_t(pooled_p.T, W1tp, b1.reshape(-1, 1), W2.T, b2.reshape(-1, 1))
    return out_t.T


# speedup vs baseline: 1.0181x; 1.0162x over previous
import functools

import jax
import jax.numpy as jnp
from jax import lax
from jax.experimental import pallas as pl
from jax.experimental.pallas import tpu as pltpu
from jax.experimental.pallas import tpu_sc as plsc

B = 4096
L = 200
E = 64
W = E // 2
VOCAB = 100000

NC = 2
NS = 16
NW = NC * NS

ROWS_PER_W = B // NW
CH = 4
N_CHUNKS = ROWS_PER_W // CH
HALF = L // 2
TOK_CH = CH * L

VB = 1024
NVB = (VOCAB + VB - 1) // VB
VPAD = NVB * VB


def _conv_body(tt_ref, o_ref):
    bound = VOCAB - pl.program_id(0) * VB
    col = lax.broadcasted_iota(jnp.int32, (E, VB), 1)
    x = jnp.where(col < bound, tt_ref[...], 0.0)
    tb = x.T.astype(jnp.bfloat16)
    lo = lax.bitcast_convert_type(tb[:, :W], jnp.uint16)
    hi = lax.bitcast_convert_type(tb[:, W:], jnp.uint16)
    w32 = (hi.astype(jnp.uint32) << 16) | lo.astype(jnp.uint32)
    words = lax.bitcast_convert_type(w32, jnp.int32)
    for m in range(4):
        o_ref[:, pl.ds(m * W, W)] = words[m * (VB // 4):(m + 1) * (VB // 4), :]


def _convert_table(table_t):
    packed = pl.pallas_call(
        _conv_body,
        grid=(NVB,),
        in_specs=[pl.BlockSpec((E, VB), lambda i: (0, i))],
        out_specs=pl.BlockSpec((VB // 4, 2 * E), lambda i: (i, 0)),
        out_shape=jax.ShapeDtypeStruct((VPAD // 4, 2 * E), jnp.int32),
    )(table_t)
    return packed.reshape(VPAD, W)


_PERM = (list(range(0, 16)) + list(range(32, 48))
         + list(range(16, 32)) + list(range(48, 64)))


GGRP = 40
NGRP = TOK_CH // GGRP
TOK_W = ROWS_PER_W * L


def _pool_body(texts_hbm, table_hbm, out_hbm, idx_v, rows_v, acc_v,
               sem0, sem1):
    wid = lax.axis_index("s") * NC + lax.axis_index("c")
    sems = (sem0, sem1)

    pltpu.sync_copy(texts_hbm.at[pl.ds(wid * TOK_W, TOK_W)], idx_v)

    def remap_body(t, carry):
        v = idx_v[pl.ds(16 * t, 16)]
        r = (lax.bitwise_and(v, jnp.int32(-1024))
             | lax.shift_left(lax.bitwise_and(v, jnp.int32(255)), 2)
             | lax.bitwise_and(lax.shift_right_logical(v, 8), jnp.int32(3)))
        idx_v[pl.ds(16 * t, 16)] = r
        return carry

    lax.fori_loop(0, TOK_W // 16, remap_body, 0)

    def stage(s, g):
        for j in range(NGRP):
            pltpu.async_copy(
                table_hbm.at[idx_v.at[pl.ds(g * TOK_CH + j * GGRP, GGRP)]],
                rows_v.at[s, pl.ds(j * GGRP, GGRP)],
                sems[s])

    def drain(s):
        pltpu.make_async_copy(
            table_hbm.at[pl.ds(0, TOK_CH)], rows_v.at[s], sems[s]).wait()

    def reduce_store(s, g):
        row_base = wid * ROWS_PER_W + g * CH
        for r in range(CH):
            def tok_body(t, acc, r=r):
                new = list(acc)
                for u in range(2):
                    base = r * L + 2 * t + u
                    for h in range(2):
                        w = rows_v[s, base, pl.ds(16 * h, 16)]
                        ev = plsc.bitcast(lax.shift_left(w, 16), jnp.float32)
                        od = plsc.bitcast(
                            lax.bitwise_and(w, jnp.int32(-65536)), jnp.float32)
                        new[2 * h] = new[2 * h] + ev
                        new[2 * h + 1] = new[2 * h + 1] + od
                return tuple(new)
            acc = lax.fori_loop(
                0, L // 2, tok_body,
                tuple(jnp.zeros((16,), jnp.float32) for _ in range(4)))
            for c in range(4):
                acc_v[r, pl.ds(c * 16, 16)] = acc[c] * (1.0 / L)
        pltpu.sync_copy(acc_v, out_hbm.at[pl.ds(row_base, CH)])

    stage(0, 0)

    def pair_body(i, carry):
        g0 = 2 * i
        stage(1, g0 + 1)
        drain(0)
        reduce_store(0, g0)

        @pl.when(g0 + 2 < N_CHUNKS)
        def _():
            stage(0, g0 + 2)

        drain(1)
        reduce_store(1, g0 + 1)
        return carry

    lax.fori_loop(0, N_CHUNKS // 2, pair_body, 0)


_pool = functools.partial(
    pl.kernel,
    out_type=jax.ShapeDtypeStruct((B, E), jnp.float32),
    mesh=plsc.VectorSubcoreMesh(core_axis_name="c", subcore_axis_name="s"),
    compiler_params=pltpu.CompilerParams(use_tc_tiling_on_sc=False,
                                         needs_layout_passes=False),
    scratch_types=[
        pltpu.VMEM((TOK_W,), jnp.int32),
        pltpu.VMEM((2, TOK_CH, W), jnp.int32),
        pltpu.VMEM((CH, E), jnp.float32),
        pltpu.SemaphoreType.DMA,
        pltpu.SemaphoreType.DMA,
    ],
)(_pool_body)


def _mlp_body(pt_ref, w1t_ref, b1_ref, w2t_ref, b2_ref, ot_ref):
    ht = jnp.dot(w1t_ref[...], pt_ref[...], preferred_element_type=jnp.float32)
    ht = jnp.maximum(ht + b1_ref[...], 0.0).astype(jnp.bfloat16)
    ot_ref[...] = (jnp.dot(w2t_ref[...], ht, preferred_element_type=jnp.float32)
                   + b2_ref[...])


def _mlp_t(pooled_t, W1t, b1c, W2t, b2c):
    BM = 512
    H = W1t.shape[0]
    C = W2t.shape[0]
    return pl.pallas_call(
        _mlp_body,
        grid=(B // BM,),
        in_specs=[
            pl.BlockSpec((E, BM), lambda i: (0, i)),
            pl.BlockSpec((H, E), lambda i: (0, 0)),
            pl.BlockSpec((H, 1), lambda i: (0, 0)),
            pl.BlockSpec((C, H), lambda i: (0, 0)),
            pl.BlockSpec((C, 1), lambda i: (0, 0)),
        ],
        out_specs=pl.BlockSpec((C, BM), lambda i: (0, i)),
        out_shape=jax.ShapeDtypeStruct((C, B), jnp.float32),
    )(pooled_t, W1t, b1c, W2t, b2c)


def kernel(texts, table, W1, b1, W2, b2):
    texts2 = texts.reshape(-1).astype(jnp.int32)
    pooled_p = _pool(texts2, _convert_table(table.T))
    W1tp = W1.T[:, jnp.array(_PERM)]
    out_t = _mlp_t(pooled_p.T, W1tp, b1.reshape(-1, 1),
                   W2.T.astype(jnp.bfloat16), b2.reshape(-1, 1))
    return out_t.T


# speedup vs baseline: 1.1621x; 1.1415x over previous
import functools

import jax
import jax.numpy as jnp
from jax import lax
from jax.experimental import pallas as pl
from jax.experimental.pallas import tpu as pltpu
from jax.experimental.pallas import tpu_sc as plsc

B = 4096
L = 200
E = 64
W = E // 2
VOCAB = 100000

NC = 2
NS = 16
NW = NC * NS

ROWS_PER_W = B // NW
CH = 4
N_CHUNKS = ROWS_PER_W // CH
HALF = L // 2
TOK_CH = CH * L

VB = 2048
NVB = (VOCAB + VB - 1) // VB
VPAD = NVB * VB


def _conv_body(tt_ref, o_ref):
    bound = VOCAB - pl.program_id(0) * VB
    col = lax.broadcasted_iota(jnp.int32, (E, VB), 1)
    x = jnp.where(col < bound, tt_ref[...], 0.0)
    tb = x.T.astype(jnp.bfloat16)
    lo = lax.bitcast_convert_type(tb[:, :W], jnp.uint16)
    hi = lax.bitcast_convert_type(tb[:, W:], jnp.uint16)
    w32 = (hi.astype(jnp.uint32) << 16) | lo.astype(jnp.uint32)
    words = lax.bitcast_convert_type(w32, jnp.int32)
    for m in range(4):
        o_ref[:, pl.ds(m * W, W)] = words[m * (VB // 4):(m + 1) * (VB // 4), :]


def _convert_table(table_t):
    packed = pl.pallas_call(
        _conv_body,
        grid=(NVB,),
        in_specs=[pl.BlockSpec((E, VB), lambda i: (0, i))],
        out_specs=pl.BlockSpec((VB // 4, 2 * E), lambda i: (i, 0)),
        out_shape=jax.ShapeDtypeStruct((VPAD // 4, 2 * E), jnp.int32),
    )(table_t)
    return packed.reshape(VPAD, W)


_PERM = (list(range(0, 16)) + list(range(32, 48))
         + list(range(16, 32)) + list(range(48, 64)))


GGRP = 40
NGRP = TOK_CH // GGRP
TOK_W = ROWS_PER_W * L


def _pool_body(texts_hbm, table_hbm, out_hbm, idx_v, rows_v, acc_v,
               sem0, sem1):
    wid = lax.axis_index("s") * NC + lax.axis_index("c")
    sems = (sem0, sem1)

    pltpu.sync_copy(texts_hbm.at[pl.ds(wid * TOK_W, TOK_W)], idx_v)

    def remap_body(t, carry):
        v = idx_v[pl.ds(16 * t, 16)]
        r = (lax.bitwise_and(v, jnp.int32(-VB))
             | lax.shift_left(lax.bitwise_and(v, jnp.int32(VB // 4 - 1)), 2)
             | lax.bitwise_and(
                 lax.shift_right_logical(v, (VB // 4).bit_length() - 1),
                 jnp.int32(3)))
        idx_v[pl.ds(16 * t, 16)] = r
        return carry

    lax.fori_loop(0, TOK_W // 16, remap_body, 0)

    def stage(s, g):
        for j in range(NGRP):
            pltpu.async_copy(
                table_hbm.at[idx_v.at[pl.ds(g * TOK_CH + j * GGRP, GGRP)]],
                rows_v.at[s, pl.ds(j * GGRP, GGRP)],
                sems[s])

    def drain(s):
        pltpu.make_async_copy(
            table_hbm.at[pl.ds(0, TOK_CH)], rows_v.at[s], sems[s]).wait()

    def reduce_store(s, g):
        row_base = wid * ROWS_PER_W + g * CH
        for r in range(CH):
            def tok_body(t, acc, r=r):
                new = list(acc)
                for u in range(2):
                    base = r * L + 2 * t + u
                    for h in range(2):
                        w = rows_v[s, base, pl.ds(16 * h, 16)]
                        ev = plsc.bitcast(lax.shift_left(w, 16), jnp.float32)
                        od = plsc.bitcast(
                            lax.bitwise_and(w, jnp.int32(-65536)), jnp.float32)
                        new[2 * h] = new[2 * h] + ev
                        new[2 * h + 1] = new[2 * h + 1] + od
                return tuple(new)
            acc = lax.fori_loop(
                0, L // 2, tok_body,
                tuple(jnp.zeros((16,), jnp.float32) for _ in range(4)))
            for c in range(4):
                acc_v[r, pl.ds(c * 16, 16)] = acc[c] * (1.0 / L)
        pltpu.sync_copy(acc_v, out_hbm.at[pl.ds(row_base, CH)])

    stage(0, 0)

    def pair_body(i, carry):
        g0 = 2 * i
        stage(1, g0 + 1)
        drain(0)
        reduce_store(0, g0)

        @pl.when(g0 + 2 < N_CHUNKS)
        def _():
            stage(0, g0 + 2)

        drain(1)
        reduce_store(1, g0 + 1)
        return carry

    lax.fori_loop(0, N_CHUNKS // 2, pair_body, 0)


_pool = functools.partial(
    pl.kernel,
    out_type=jax.ShapeDtypeStruct((B, E), jnp.float32),
    mesh=plsc.VectorSubcoreMesh(core_axis_name="c", subcore_axis_name="s"),
    compiler_params=pltpu.CompilerParams(use_tc_tiling_on_sc=False,
                                         needs_layout_passes=False),
    scratch_types=[
        pltpu.VMEM((TOK_W,), jnp.int32),
        pltpu.VMEM((2, TOK_CH, W), jnp.int32),
        pltpu.VMEM((CH, E), jnp.float32),
        pltpu.SemaphoreType.DMA,
        pltpu.SemaphoreType.DMA,
    ],
)(_pool_body)


def _mlp_body(pt_ref, w1t_ref, b1_ref, w2t_ref, b2_ref, ot_ref):
    ht = jnp.dot(w1t_ref[...], pt_ref[...], preferred_element_type=jnp.float32)
    ht = jnp.maximum(ht + b1_ref[...], 0.0).astype(jnp.bfloat16)
    ot_ref[...] = (jnp.dot(w2t_ref[...], ht, preferred_element_type=jnp.float32)
                   + b2_ref[...])


def _mlp_t(pooled_t, W1t, b1c, W2t, b2c):
    BM = 512
    H = W1t.shape[0]
    C = W2t.shape[0]
    return pl.pallas_call(
        _mlp_body,
        grid=(B // BM,),
        in_specs=[
            pl.BlockSpec((E, BM), lambda i: (0, i)),
            pl.BlockSpec((H, E), lambda i: (0, 0)),
            pl.BlockSpec((H, 1), lambda i: (0, 0)),
            pl.BlockSpec((C, H), lambda i: (0, 0)),
            pl.BlockSpec((C, 1), lambda i: (0, 0)),
        ],
        out_specs=pl.BlockSpec((C, BM), lambda i: (0, i)),
        out_shape=jax.ShapeDtypeStruct((C, B), jnp.float32),
    )(pooled_t, W1t, b1c, W2t, b2c)


def kernel(texts, table, W1, b1, W2, b2):
    texts2 = texts.reshape(-1).astype(jnp.int32)
    pooled_p = _pool(texts2, _convert_table(table.T))
    W1tp = W1.T[:, jnp.array(_PERM)]
    out_t = _mlp_t(pooled_p.T, W1tp, b1.reshape(-1, 1),
                   W2.T.astype(jnp.bfloat16), b2.reshape(-1, 1))
    return out_t.T


# speedup vs baseline: 1.2471x; 1.0731x over previous
import functools

import jax
import jax.numpy as jnp
from jax import lax
from jax.experimental import pallas as pl
from jax.experimental.pallas import tpu as pltpu
from jax.experimental.pallas import tpu_sc as plsc

B = 4096
L = 200
E = 64
W = E // 2
VOCAB = 100000

NC = 2
NS = 16
NW = NC * NS

ROWS_PER_W = B // NW
CH = 4
N_CHUNKS = ROWS_PER_W // CH
HALF = L // 2
TOK_CH = CH * L

VB = 4096
NVB = (VOCAB + VB - 1) // VB
VPAD = NVB * VB


def _conv_body(tt_ref, o_ref):
    bound = VOCAB - pl.program_id(0) * VB
    col = lax.broadcasted_iota(jnp.int32, (E, VB), 1)
    x = jnp.where(col < bound, tt_ref[...], 0.0)
    tb = x.T.astype(jnp.bfloat16)
    lo = lax.bitcast_convert_type(tb[:, :W], jnp.uint16)
    hi = lax.bitcast_convert_type(tb[:, W:], jnp.uint16)
    w32 = (hi.astype(jnp.uint32) << 16) | lo.astype(jnp.uint32)
    words = lax.bitcast_convert_type(w32, jnp.int32)
    for m in range(4):
        o_ref[:, pl.ds(m * W, W)] = words[m * (VB // 4):(m + 1) * (VB // 4), :]


def _convert_table(table_t):
    packed = pl.pallas_call(
        _conv_body,
        grid=(NVB,),
        in_specs=[pl.BlockSpec((E, VB), lambda i: (0, i))],
        out_specs=pl.BlockSpec((VB // 4, 2 * E), lambda i: (i, 0)),
        out_shape=jax.ShapeDtypeStruct((VPAD // 4, 2 * E), jnp.int32),
    )(table_t)
    return packed.reshape(VPAD, W)


_PERM = (list(range(0, 16)) + list(range(32, 48))
         + list(range(16, 32)) + list(range(48, 64)))


GGRP = 40
NGRP = TOK_CH // GGRP
TOK_W = ROWS_PER_W * L


def _pool_body(texts_hbm, table_hbm, out_hbm, idx_v, rows_v, acc_v,
               sem0, sem1):
    wid = lax.axis_index("s") * NC + lax.axis_index("c")
    sems = (sem0, sem1)

    pltpu.sync_copy(texts_hbm.at[pl.ds(wid * TOK_W, TOK_W)], idx_v)

    def remap_body(t, carry):
        v = idx_v[pl.ds(16 * t, 16)]
        r = (lax.bitwise_and(v, jnp.int32(-VB))
             | lax.shift_left(lax.bitwise_and(v, jnp.int32(VB // 4 - 1)), 2)
             | lax.bitwise_and(
                 lax.shift_right_logical(v, (VB // 4).bit_length() - 1),
                 jnp.int32(3)))
        idx_v[pl.ds(16 * t, 16)] = r
        return carry

    lax.fori_loop(0, TOK_W // 16, remap_body, 0)

    def stage(s, g):
        for j in range(NGRP):
            pltpu.async_copy(
                table_hbm.at[idx_v.at[pl.ds(g * TOK_CH + j * GGRP, GGRP)]],
                rows_v.at[s, pl.ds(j * GGRP, GGRP)],
                sems[s])

    def drain(s):
        pltpu.make_async_copy(
            table_hbm.at[pl.ds(0, TOK_CH)], rows_v.at[s], sems[s]).wait()

    def reduce_store(s, g):
        row_base = wid * ROWS_PER_W + g * CH
        for r in range(CH):
            def tok_body(t, acc, r=r):
                new = list(acc)
                for u in range(2):
                    base = r * L + 2 * t + u
                    for h in range(2):
                        w = rows_v[s, base, pl.ds(16 * h, 16)]
                        ev = plsc.bitcast(lax.shift_left(w, 16), jnp.float32)
                        od = plsc.bitcast(
                            lax.bitwise_and(w, jnp.int32(-65536)), jnp.float32)
                        new[2 * h] = new[2 * h] + ev
                        new[2 * h + 1] = new[2 * h + 1] + od
                return tuple(new)
            acc = lax.fori_loop(
                0, L // 2, tok_body,
                tuple(jnp.zeros((16,), jnp.float32) for _ in range(4)))
            for c in range(4):
                acc_v[r, pl.ds(c * 16, 16)] = acc[c] * (1.0 / L)
        pltpu.sync_copy(acc_v, out_hbm.at[pl.ds(row_base, CH)])

    stage(0, 0)

    def pair_body(i, carry):
        g0 = 2 * i
        stage(1, g0 + 1)
        drain(0)
        reduce_store(0, g0)

        @pl.when(g0 + 2 < N_CHUNKS)
        def _():
            stage(0, g0 + 2)

        drain(1)
        reduce_store(1, g0 + 1)
        return carry

    lax.fori_loop(0, N_CHUNKS // 2, pair_body, 0)


_pool = functools.partial(
    pl.kernel,
    out_type=jax.ShapeDtypeStruct((B, E), jnp.float32),
    mesh=plsc.VectorSubcoreMesh(core_axis_name="c", subcore_axis_name="s"),
    compiler_params=pltpu.CompilerParams(use_tc_tiling_on_sc=False,
                                         needs_layout_passes=False),
    scratch_types=[
        pltpu.VMEM((TOK_W,), jnp.int32),
        pltpu.VMEM((2, TOK_CH, W), jnp.int32),
        pltpu.VMEM((CH, E), jnp.float32),
        pltpu.SemaphoreType.DMA,
        pltpu.SemaphoreType.DMA,
    ],
)(_pool_body)


def _mlp_body(pt_ref, w1t_ref, b1_ref, w2t_ref, b2_ref, ot_ref):
    ht = jnp.dot(w1t_ref[...], pt_ref[...], preferred_element_type=jnp.float32)
    ht = jnp.maximum(ht + b1_ref[...], 0.0).astype(jnp.bfloat16)
    ot_ref[...] = (jnp.dot(w2t_ref[...], ht, preferred_element_type=jnp.float32)
                   + b2_ref[...])


def _mlp_t(pooled_t, W1t, b1c, W2t, b2c):
    BM = 512
    H = W1t.shape[0]
    C = W2t.shape[0]
    return pl.pallas_call(
        _mlp_body,
        grid=(B // BM,),
        in_specs=[
            pl.BlockSpec((E, BM), lambda i: (0, i)),
            pl.BlockSpec((H, E), lambda i: (0, 0)),
            pl.BlockSpec((H, 1), lambda i: (0, 0)),
            pl.BlockSpec((C, H), lambda i: (0, 0)),
            pl.BlockSpec((C, 1), lambda i: (0, 0)),
        ],
        out_specs=pl.BlockSpec((C, BM), lambda i: (0, i)),
        out_shape=jax.ShapeDtypeStruct((C, B), jnp.float32),
    )(pooled_t, W1t, b1c, W2t, b2c)


def kernel(texts, table, W1, b1, W2, b2):
    texts2 = texts.reshape(-1).astype(jnp.int32)
    pooled_p = _pool(texts2, _convert_table(table.T))
    W1tp = W1.T[:, jnp.array(_PERM)]
    out_t = _mlp_t(pooled_p.T, W1tp, b1.reshape(-1, 1),
                   W2.T.astype(jnp.bfloat16), b2.reshape(-1, 1))
    return out_t.T
